# Initial kernel scaffold; baseline (speedup 1.0000x reference)
import jax
import jax.numpy as jnp
from jax.experimental import pallas as pl


def kernel(node_attr, edge_index, edge_attr, edge_sh, W1, b1, W2, b2):
    raise NotImplementedError("write your pallas kernel here")


# speedup vs baseline: 3.5926x; 3.5926x over previous
import jax
import jax.numpy as jnp
import numpy as np
from jax import lax
from jax.experimental import pallas as pl
from jax.experimental.pallas import tpu as pltpu
from jax.experimental.pallas import tpu_sc as plsc

N_NODES = 10000
N_EDGES = 320000
F_IN = 16
F_HID = 32
F_W = 256
ALPHA = 1.0 / np.sqrt(16.0)

NC = 2
NS = 16
NW = NC * NS
EPW = N_EDGES // NW
CH = 128
NFULL = EPW // CH
TAIL = EPW - NFULL * CH
NPAD = 10240
STRIPE = NPAD // NS

def _mesh():
    return plsc.VectorSubcoreMesh(core_axis_name="c", subcore_axis_name="s",
                                  num_cores=NC, num_subcores=NS)


def _gather_body(nodes_hbm, dst_hbm, x_hbm, idx_v, rows_v, idx_t, rows_t, sem):
    c = lax.axis_index("c")
    s = lax.axis_index("s")
    wid = s * NC + c
    base = wid * EPW

    def chunk(off, idx_ref, rows_ref):
        n = idx_ref.shape[0]
        pltpu.sync_copy(dst_hbm.at[pl.ds(off, n)], idx_ref)
        pltpu.async_copy(nodes_hbm.at[idx_ref], rows_ref, sem).wait()
        pltpu.sync_copy(rows_ref, x_hbm.at[pl.ds(off, n)])

    def body(j, carry):
        chunk(base + j * CH, idx_v, rows_v)
        return carry

    lax.fori_loop(0, NFULL, body, 0)
    chunk(base + NFULL * CH, idx_t, rows_t)


@jax.jit
def _sc_gather(node_attr, edge_dst):
    return pl.kernel(
        _gather_body,
        out_type=jax.ShapeDtypeStruct((N_EDGES, F_IN), jnp.float32),
        mesh=_mesh(),
        compiler_params=pltpu.CompilerParams(use_tc_tiling_on_sc=False),
        scratch_types=[
            pltpu.VMEM((CH,), jnp.int32),
            pltpu.VMEM((CH, F_IN), jnp.float32),
            pltpu.VMEM((TAIL,), jnp.int32),
            pltpu.VMEM((TAIL, F_IN), jnp.float32),
            pltpu.SemaphoreType.DMA,
        ],
    )(node_attr, edge_dst)


def _scatter_body(tp_hbm, src_hbm, psum_hbm, pcnt_hbm,
                  idx_v, rows_v, ones_v, idx_t, rows_t, ones_t,
                  zrow_v, zcnt_v, acc_sh, cnt_sh, sem):
    c = lax.axis_index("c")
    s = lax.axis_index("s")
    wid = s * NC + c
    base = wid * EPW

    onesv = jnp.ones((16,), jnp.float32)
    zerov = jnp.zeros((16,), jnp.float32)
    for i in range(CH // 16):
        ones_v[pl.ds(i * 16, 16)] = onesv
    ones_t[...] = onesv

    def zrow(i, carry):
        zrow_v[i, :] = zerov
        return carry
    lax.fori_loop(0, STRIPE, zrow, 0)

    def zcnt(i, carry):
        zcnt_v[pl.ds(i * 16, 16)] = zerov
        return carry
    lax.fori_loop(0, STRIPE // 16, zcnt, 0)

    pltpu.sync_copy(zrow_v, acc_sh.at[pl.ds(s * STRIPE, STRIPE)])
    pltpu.sync_copy(zcnt_v, cnt_sh.at[pl.ds(s * STRIPE, STRIPE)])
    plsc.subcore_barrier()

    def chunk(off, idx_ref, rows_ref, ones_ref):
        n = idx_ref.shape[0]
        pltpu.sync_copy(src_hbm.at[pl.ds(off, n)], idx_ref)
        pltpu.sync_copy(tp_hbm.at[pl.ds(off, n)], rows_ref)
        pltpu.sync_copy(rows_ref, acc_sh.at[idx_ref], add=True)
        pltpu.sync_copy(ones_ref, cnt_sh.at[idx_ref], add=True)

    def body(j, carry):
        chunk(base + j * CH, idx_v, rows_v, ones_v)
        return carry

    lax.fori_loop(0, NFULL, body, 0)
    chunk(base + NFULL * CH, idx_t, rows_t, ones_t)

    plsc.subcore_barrier()
    pltpu.sync_copy(acc_sh.at[pl.ds(s * STRIPE, STRIPE)],
                    psum_hbm.at[c, pl.ds(s * STRIPE, STRIPE)])
    pltpu.sync_copy(cnt_sh.at[pl.ds(s * STRIPE, STRIPE)],
                    pcnt_hbm.at[c, pl.ds(s * STRIPE, STRIPE)])


@jax.jit
def _sc_scatter(tp, edge_src):
    return pl.kernel(
        _scatter_body,
        out_type=(
            jax.ShapeDtypeStruct((NC, NPAD, F_IN), jnp.float32),
            jax.ShapeDtypeStruct((NC, NPAD), jnp.float32),
        ),
        mesh=_mesh(),
        compiler_params=pltpu.CompilerParams(use_tc_tiling_on_sc=False),
        scratch_types=[
            pltpu.VMEM((CH,), jnp.int32),
            pltpu.VMEM((CH, F_IN), jnp.float32),
            pltpu.VMEM((CH,), jnp.float32),
            pltpu.VMEM((TAIL,), jnp.int32),
            pltpu.VMEM((TAIL, F_IN), jnp.float32),
            pltpu.VMEM((TAIL,), jnp.float32),
            pltpu.VMEM((STRIPE, F_IN), jnp.float32),
            pltpu.VMEM((STRIPE,), jnp.float32),
            pltpu.VMEM_SHARED((NPAD, F_IN), jnp.float32),
            pltpu.VMEM_SHARED((NPAD,), jnp.float32),
            pltpu.SemaphoreType.DMA,
        ],
    )(tp, edge_src)


EB = 2000


def _edge_body(ea_ref, x_ref, y_ref, W1_ref, b1_ref, W2_ref, b2_ref,
               R_ref, S_ref, tp_ref):
    z = jnp.dot(ea_ref[...], W1_ref[...], preferred_element_type=jnp.float32)
    z = z + b1_ref[...][None, :]
    h = jnp.maximum(z, 0.0) + jnp.log1p(jnp.exp(-jnp.abs(z)))
    w = jnp.dot(h, W2_ref[...], preferred_element_type=jnp.float32)
    w = w + b2_ref[...][None, :]
    xr = jnp.dot(x_ref[...], R_ref[...], preferred_element_type=jnp.float32)
    tp = jnp.dot(xr * w, S_ref[...], preferred_element_type=jnp.float32)
    tp_ref[...] = tp * (ALPHA * y_ref[...])


@jax.jit
def _tc_edge(edge_attr, x, edge_sh, W1, b1, W2, b2):
    R = jnp.asarray(np.kron(np.eye(F_IN, dtype=np.float32),
                            np.ones((1, F_IN), np.float32)))
    S = jnp.asarray(np.kron(np.ones((F_IN, 1), np.float32),
                            np.eye(F_IN, dtype=np.float32)))
    grid = N_EDGES // EB
    return pl.pallas_call(
        _edge_body,
        grid=(grid,),
        in_specs=[
            pl.BlockSpec((EB, F_HID), lambda i: (i, 0)),
            pl.BlockSpec((EB, F_IN), lambda i: (i, 0)),
            pl.BlockSpec((EB, 1), lambda i: (i, 0)),
            pl.BlockSpec((F_HID, F_HID), lambda i: (0, 0)),
            pl.BlockSpec((F_HID,), lambda i: (0,)),
            pl.BlockSpec((F_HID, F_W), lambda i: (0, 0)),
            pl.BlockSpec((F_W,), lambda i: (0,)),
            pl.BlockSpec((F_IN, F_W), lambda i: (0, 0)),
            pl.BlockSpec((F_W, F_IN), lambda i: (0, 0)),
        ],
        out_specs=pl.BlockSpec((EB, F_IN), lambda i: (i, 0)),
        out_shape=jax.ShapeDtypeStruct((N_EDGES, F_IN), jnp.float32),
    )(edge_attr, x, edge_sh, W1, b1, W2, b2, R, S)


def _combine_body(p0_ref, p1_ref, c0_ref, c1_ref, na_ref, out_ref):
    cnt = jnp.maximum(c0_ref[...] + c1_ref[...], 1.0)
    out_ref[...] = (p0_ref[...] + p1_ref[...]) / cnt + na_ref[...]


@jax.jit
def _tc_combine(psum, pcnt, node_attr):
    p0 = psum[0, :N_NODES]
    p1 = psum[1, :N_NODES]
    c0 = pcnt[0, :N_NODES].reshape(N_NODES, 1)
    c1 = pcnt[1, :N_NODES].reshape(N_NODES, 1)
    return pl.pallas_call(
        _combine_body,
        out_shape=jax.ShapeDtypeStruct((N_NODES, F_IN), jnp.float32),
    )(p0, p1, c0, c1, node_attr)


def kernel(node_attr, edge_index, edge_attr, edge_sh, W1, b1, W2, b2):
    edge_src = edge_index[0]
    edge_dst = edge_index[1]
    x = _sc_gather(node_attr, edge_dst)
    tp = _tc_edge(edge_attr, x, edge_sh, W1, b1, W2, b2)
    psum, pcnt = _sc_scatter(tp, edge_src)
    return _tc_combine(psum, pcnt, node_attr)


# speedup vs baseline: 3.9413x; 1.0971x over previous
import jax
import jax.numpy as jnp
import numpy as np
from jax import lax
from jax.experimental import pallas as pl
from jax.experimental.pallas import tpu as pltpu
from jax.experimental.pallas import tpu_sc as plsc

N_NODES = 10000
N_EDGES = 320000
F_IN = 16
F_HID = 32
F_W = 256
ALPHA = 1.0 / np.sqrt(16.0)

NC = 2
NS = 16
NW = NC * NS
EPW = N_EDGES // NW
CH = 2000
NCHUNK = EPW // CH
NPAD = 10240
STRIPE = NPAD // NS

def _mesh():
    return plsc.VectorSubcoreMesh(core_axis_name="c", subcore_axis_name="s",
                                  num_cores=NC, num_subcores=NS)


def _gather_body(nodes_hbm, dst_hbm, x_hbm, idx_v, rows_v, sem):
    c = lax.axis_index("c")
    s = lax.axis_index("s")
    wid = s * NC + c
    base = wid * EPW

    def body(j, carry):
        off = base + j * CH
        pltpu.sync_copy(dst_hbm.at[pl.ds(off, CH)], idx_v)
        pltpu.async_copy(nodes_hbm.at[idx_v], rows_v, sem).wait()
        pltpu.sync_copy(rows_v, x_hbm.at[pl.ds(off, CH)])
        return carry

    lax.fori_loop(0, NCHUNK, body, 0)


@jax.jit
def _sc_gather(node_attr, edge_dst):
    return pl.kernel(
        _gather_body,
        out_type=jax.ShapeDtypeStruct((N_EDGES, F_IN), jnp.float32),
        mesh=_mesh(),
        compiler_params=pltpu.CompilerParams(use_tc_tiling_on_sc=False),
        scratch_types=[
            pltpu.VMEM((CH,), jnp.int32),
            pltpu.VMEM((CH, F_IN), jnp.float32),
            pltpu.SemaphoreType.DMA,
        ],
    )(node_attr, edge_dst)


def _scatter_body(tp_hbm, src_hbm, psum_hbm, pcnt_hbm,
                  idx_v, rows_v, ones_v, zrow_v, zcnt_v,
                  acc_sh, cnt_sh, sem):
    c = lax.axis_index("c")
    s = lax.axis_index("s")
    wid = s * NC + c
    base = wid * EPW

    onesv = jnp.ones((16,), jnp.float32)
    zerov = jnp.zeros((16,), jnp.float32)

    def fones(i, carry):
        ones_v[pl.ds(i * 16, 16)] = onesv
        return carry
    lax.fori_loop(0, CH // 16, fones, 0)

    def zrow(i, carry):
        zrow_v[i, :] = zerov
        return carry
    lax.fori_loop(0, STRIPE, zrow, 0)

    def zcnt(i, carry):
        zcnt_v[pl.ds(i * 16, 16)] = zerov
        return carry
    lax.fori_loop(0, STRIPE // 16, zcnt, 0)

    pltpu.sync_copy(zrow_v, acc_sh.at[pl.ds(s * STRIPE, STRIPE)])
    pltpu.sync_copy(zcnt_v, cnt_sh.at[pl.ds(s * STRIPE, STRIPE)])
    plsc.subcore_barrier()

    def body(j, carry):
        off = base + j * CH
        pltpu.sync_copy(src_hbm.at[pl.ds(off, CH)], idx_v)
        pltpu.sync_copy(tp_hbm.at[pl.ds(off, CH)], rows_v)
        pltpu.sync_copy(rows_v, acc_sh.at[idx_v], add=True)
        pltpu.sync_copy(ones_v, cnt_sh.at[idx_v], add=True)
        return carry

    lax.fori_loop(0, NCHUNK, body, 0)

    plsc.subcore_barrier()
    pltpu.sync_copy(acc_sh.at[pl.ds(s * STRIPE, STRIPE)],
                    psum_hbm.at[c, pl.ds(s * STRIPE, STRIPE)])
    pltpu.sync_copy(cnt_sh.at[pl.ds(s * STRIPE, STRIPE)],
                    pcnt_hbm.at[c, pl.ds(s * STRIPE, STRIPE)])


@jax.jit
def _sc_scatter(tp, edge_src):
    return pl.kernel(
        _scatter_body,
        out_type=(
            jax.ShapeDtypeStruct((NC, NPAD, F_IN), jnp.float32),
            jax.ShapeDtypeStruct((NC, NPAD), jnp.float32),
        ),
        mesh=_mesh(),
        compiler_params=pltpu.CompilerParams(use_tc_tiling_on_sc=False),
        scratch_types=[
            pltpu.VMEM((CH,), jnp.int32),
            pltpu.VMEM((CH, F_IN), jnp.float32),
            pltpu.VMEM((CH,), jnp.float32),
            pltpu.VMEM((STRIPE, F_IN), jnp.float32),
            pltpu.VMEM((STRIPE,), jnp.float32),
            pltpu.VMEM_SHARED((NPAD, F_IN), jnp.float32),
            pltpu.VMEM_SHARED((NPAD,), jnp.float32),
            pltpu.SemaphoreType.DMA,
        ],
    )(tp, edge_src)


EB = 2000


def _edge_body(ea_ref, x_ref, y_ref, W1_ref, b1_ref, W2_ref, b2_ref,
               R_ref, S_ref, tp_ref):
    z = jnp.dot(ea_ref[...], W1_ref[...], preferred_element_type=jnp.float32)
    z = z + b1_ref[...][None, :]
    h = jnp.maximum(z, 0.0) + jnp.log1p(jnp.exp(-jnp.abs(z)))
    w = jnp.dot(h, W2_ref[...], preferred_element_type=jnp.float32)
    w = w + b2_ref[...][None, :]
    xr = jnp.dot(x_ref[...], R_ref[...], preferred_element_type=jnp.float32)
    tp = jnp.dot(xr * w, S_ref[...], preferred_element_type=jnp.float32)
    tp_ref[...] = tp * (ALPHA * y_ref[...])


@jax.jit
def _tc_edge(edge_attr, x, edge_sh, W1, b1, W2, b2):
    R = jnp.asarray(np.kron(np.eye(F_IN, dtype=np.float32),
                            np.ones((1, F_IN), np.float32)))
    S = jnp.asarray(np.kron(np.ones((F_IN, 1), np.float32),
                            np.eye(F_IN, dtype=np.float32)))
    grid = N_EDGES // EB
    return pl.pallas_call(
        _edge_body,
        grid=(grid,),
        in_specs=[
            pl.BlockSpec((EB, F_HID), lambda i: (i, 0)),
            pl.BlockSpec((EB, F_IN), lambda i: (i, 0)),
            pl.BlockSpec((EB, 1), lambda i: (i, 0)),
            pl.BlockSpec((F_HID, F_HID), lambda i: (0, 0)),
            pl.BlockSpec((F_HID,), lambda i: (0,)),
            pl.BlockSpec((F_HID, F_W), lambda i: (0, 0)),
            pl.BlockSpec((F_W,), lambda i: (0,)),
            pl.BlockSpec((F_IN, F_W), lambda i: (0, 0)),
            pl.BlockSpec((F_W, F_IN), lambda i: (0, 0)),
        ],
        out_specs=pl.BlockSpec((EB, F_IN), lambda i: (i, 0)),
        out_shape=jax.ShapeDtypeStruct((N_EDGES, F_IN), jnp.float32),
    )(edge_attr, x, edge_sh, W1, b1, W2, b2, R, S)


def _combine_body(p0_ref, p1_ref, c0_ref, c1_ref, na_ref, out_ref):
    cnt = jnp.maximum(c0_ref[...] + c1_ref[...], 1.0)
    out_ref[...] = (p0_ref[...] + p1_ref[...]) / cnt + na_ref[...]


@jax.jit
def _tc_combine(psum, pcnt, node_attr):
    p0 = psum[0, :N_NODES]
    p1 = psum[1, :N_NODES]
    c0 = pcnt[0, :N_NODES].reshape(N_NODES, 1)
    c1 = pcnt[1, :N_NODES].reshape(N_NODES, 1)
    return pl.pallas_call(
        _combine_body,
        out_shape=jax.ShapeDtypeStruct((N_NODES, F_IN), jnp.float32),
    )(p0, p1, c0, c1, node_attr)


def kernel(node_attr, edge_index, edge_attr, edge_sh, W1, b1, W2, b2):
    edge_src = edge_index[0]
    edge_dst = edge_index[1]
    x = _sc_gather(node_attr, edge_dst)
    tp = _tc_edge(edge_attr, x, edge_sh, W1, b1, W2, b2)
    psum, pcnt = _sc_scatter(tp, edge_src)
    return _tc_combine(psum, pcnt, node_attr)


# speedup vs baseline: 3.9880x; 1.0119x over previous
import jax
import jax.numpy as jnp
import numpy as np
from jax import lax
from jax.experimental import pallas as pl
from jax.experimental.pallas import tpu as pltpu
from jax.experimental.pallas import tpu_sc as plsc

N_NODES = 10000
N_EDGES = 320000
F_IN = 16
F_HID = 32
F_W = 256
ALPHA = 1.0 / np.sqrt(16.0)

NC = 2
NS = 16
NW = NC * NS
EPW = N_EDGES // NW
CH = 2000
NCHUNK = EPW // CH
NPAD = 10240
STRIPE = NPAD // NS

def _mesh():
    return plsc.VectorSubcoreMesh(core_axis_name="c", subcore_axis_name="s",
                                  num_cores=NC, num_subcores=NS)


def _gather_body(nodes_hbm, dst_hbm, x_hbm,
                 idx0, idx1, rows0, rows1,
                 isem0, isem1, gsem, wsem0, wsem1):
    c = lax.axis_index("c")
    s = lax.axis_index("s")
    wid = s * NC + c
    base = wid * EPW
    idx = (idx0, idx1)
    rows = (rows0, rows1)
    isem = (isem0, isem1)
    wsem = (wsem0, wsem1)

    ih = [None] * NCHUNK
    wh = [None] * NCHUNK
    ih[0] = pltpu.async_copy(dst_hbm.at[pl.ds(base, CH)], idx[0], isem[0])
    for j in range(NCHUNK):
        b = j % 2
        if j + 1 < NCHUNK:
            off_n = base + (j + 1) * CH
            ih[j + 1] = pltpu.async_copy(dst_hbm.at[pl.ds(off_n, CH)],
                                         idx[1 - b], isem[1 - b])
        ih[j].wait()
        if j >= 2:
            wh[j - 2].wait()
        pltpu.async_copy(nodes_hbm.at[idx[b]], rows[b], gsem).wait()
        wh[j] = pltpu.async_copy(rows[b], x_hbm.at[pl.ds(base + j * CH, CH)],
                                 wsem[b])
    wh[NCHUNK - 2].wait()
    wh[NCHUNK - 1].wait()


@jax.jit
def _sc_gather(node_attr, edge_dst):
    return pl.kernel(
        _gather_body,
        out_type=jax.ShapeDtypeStruct((N_EDGES, F_IN), jnp.float32),
        mesh=_mesh(),
        compiler_params=pltpu.CompilerParams(use_tc_tiling_on_sc=False),
        scratch_types=[
            pltpu.VMEM((CH,), jnp.int32),
            pltpu.VMEM((CH,), jnp.int32),
            pltpu.VMEM((CH, F_IN), jnp.float32),
            pltpu.VMEM((CH, F_IN), jnp.float32),
            pltpu.SemaphoreType.DMA,
            pltpu.SemaphoreType.DMA,
            pltpu.SemaphoreType.DMA,
            pltpu.SemaphoreType.DMA,
            pltpu.SemaphoreType.DMA,
        ],
    )(node_attr, edge_dst)


def _scatter_body(tp_hbm, src_hbm, psum_hbm, pcnt_hbm,
                  idx0, idx1, rows0, rows1, ones_v, zrow_v, zcnt_v,
                  acc_sh, cnt_sh, isem0, isem1, rsem0, rsem1):
    c = lax.axis_index("c")
    s = lax.axis_index("s")
    wid = s * NC + c
    base = wid * EPW
    idx = (idx0, idx1)
    rows = (rows0, rows1)
    isem = (isem0, isem1)
    rsem = (rsem0, rsem1)

    ih = [None] * NCHUNK
    rh = [None] * NCHUNK
    ih[0] = pltpu.async_copy(src_hbm.at[pl.ds(base, CH)], idx[0], isem[0])
    rh[0] = pltpu.async_copy(tp_hbm.at[pl.ds(base, CH)], rows[0], rsem[0])

    onesv = jnp.ones((16,), jnp.float32)
    zerov = jnp.zeros((16,), jnp.float32)

    def fones(i, carry):
        ones_v[pl.ds(i * 16, 16)] = onesv
        return carry
    lax.fori_loop(0, CH // 16, fones, 0)

    def zrow(i, carry):
        zrow_v[i, :] = zerov
        return carry
    lax.fori_loop(0, STRIPE, zrow, 0)

    def zcnt(i, carry):
        zcnt_v[pl.ds(i * 16, 16)] = zerov
        return carry
    lax.fori_loop(0, STRIPE // 16, zcnt, 0)

    pltpu.sync_copy(zrow_v, acc_sh.at[pl.ds(s * STRIPE, STRIPE)])
    pltpu.sync_copy(zcnt_v, cnt_sh.at[pl.ds(s * STRIPE, STRIPE)])
    plsc.subcore_barrier()

    for j in range(NCHUNK):
        b = j % 2
        if j + 1 < NCHUNK:
            off_n = base + (j + 1) * CH
            ih[j + 1] = pltpu.async_copy(src_hbm.at[pl.ds(off_n, CH)],
                                         idx[1 - b], isem[1 - b])
            rh[j + 1] = pltpu.async_copy(tp_hbm.at[pl.ds(off_n, CH)],
                                         rows[1 - b], rsem[1 - b])
        ih[j].wait()
        rh[j].wait()
        pltpu.sync_copy(rows[b], acc_sh.at[idx[b]], add=True)
        pltpu.sync_copy(ones_v, cnt_sh.at[idx[b]], add=True)

    plsc.subcore_barrier()
    pltpu.sync_copy(acc_sh.at[pl.ds(s * STRIPE, STRIPE)],
                    psum_hbm.at[c, pl.ds(s * STRIPE, STRIPE)])
    pltpu.sync_copy(cnt_sh.at[pl.ds(s * STRIPE, STRIPE)],
                    pcnt_hbm.at[c, pl.ds(s * STRIPE, STRIPE)])


@jax.jit
def _sc_scatter(tp, edge_src):
    return pl.kernel(
        _scatter_body,
        out_type=(
            jax.ShapeDtypeStruct((NC, NPAD, F_IN), jnp.float32),
            jax.ShapeDtypeStruct((NC, NPAD), jnp.float32),
        ),
        mesh=_mesh(),
        compiler_params=pltpu.CompilerParams(use_tc_tiling_on_sc=False),
        scratch_types=[
            pltpu.VMEM((CH,), jnp.int32),
            pltpu.VMEM((CH,), jnp.int32),
            pltpu.VMEM((CH, F_IN), jnp.float32),
            pltpu.VMEM((CH, F_IN), jnp.float32),
            pltpu.VMEM((CH,), jnp.float32),
            pltpu.VMEM((STRIPE, F_IN), jnp.float32),
            pltpu.VMEM((STRIPE,), jnp.float32),
            pltpu.VMEM_SHARED((NPAD, F_IN), jnp.float32),
            pltpu.VMEM_SHARED((NPAD,), jnp.float32),
            pltpu.SemaphoreType.DMA,
            pltpu.SemaphoreType.DMA,
            pltpu.SemaphoreType.DMA,
            pltpu.SemaphoreType.DMA,
        ],
    )(tp, edge_src)


EB = 2000


def _edge_body(ea_ref, x_ref, y_ref, W1_ref, b1_ref, W2_ref, b2_ref,
               R_ref, S_ref, tp_ref):
    z = jnp.dot(ea_ref[...], W1_ref[...], preferred_element_type=jnp.float32)
    z = z + b1_ref[...][None, :]
    h = jnp.maximum(z, 0.0) + jnp.log1p(jnp.exp(-jnp.abs(z)))
    w = jnp.dot(h, W2_ref[...], preferred_element_type=jnp.float32)
    w = w + b2_ref[...][None, :]
    xr = jnp.dot(x_ref[...], R_ref[...], preferred_element_type=jnp.float32)
    tp = jnp.dot(xr * w, S_ref[...], preferred_element_type=jnp.float32)
    tp_ref[...] = tp * (ALPHA * y_ref[...])


@jax.jit
def _tc_edge(edge_attr, x, edge_sh, W1, b1, W2, b2):
    R = jnp.asarray(np.kron(np.eye(F_IN, dtype=np.float32),
                            np.ones((1, F_IN), np.float32)))
    S = jnp.asarray(np.kron(np.ones((F_IN, 1), np.float32),
                            np.eye(F_IN, dtype=np.float32)))
    grid = N_EDGES // EB
    return pl.pallas_call(
        _edge_body,
        grid=(grid,),
        in_specs=[
            pl.BlockSpec((EB, F_HID), lambda i: (i, 0)),
            pl.BlockSpec((EB, F_IN), lambda i: (i, 0)),
            pl.BlockSpec((EB, 1), lambda i: (i, 0)),
            pl.BlockSpec((F_HID, F_HID), lambda i: (0, 0)),
            pl.BlockSpec((F_HID,), lambda i: (0,)),
            pl.BlockSpec((F_HID, F_W), lambda i: (0, 0)),
            pl.BlockSpec((F_W,), lambda i: (0,)),
            pl.BlockSpec((F_IN, F_W), lambda i: (0, 0)),
            pl.BlockSpec((F_W, F_IN), lambda i: (0, 0)),
        ],
        out_specs=pl.BlockSpec((EB, F_IN), lambda i: (i, 0)),
        out_shape=jax.ShapeDtypeStruct((N_EDGES, F_IN), jnp.float32),
    )(edge_attr, x, edge_sh, W1, b1, W2, b2, R, S)


def _combine_body(p0_ref, p1_ref, c0_ref, c1_ref, na_ref, out_ref):
    cnt = jnp.maximum(c0_ref[...] + c1_ref[...], 1.0)
    out_ref[...] = (p0_ref[...] + p1_ref[...]) / cnt + na_ref[...]


@jax.jit
def _tc_combine(psum, pcnt, node_attr):
    p0 = psum[0, :N_NODES]
    p1 = psum[1, :N_NODES]
    c0 = pcnt[0, :N_NODES].reshape(N_NODES, 1)
    c1 = pcnt[1, :N_NODES].reshape(N_NODES, 1)
    return pl.pallas_call(
        _combine_body,
        out_shape=jax.ShapeDtypeStruct((N_NODES, F_IN), jnp.float32),
    )(p0, p1, c0, c1, node_attr)


def kernel(node_attr, edge_index, edge_attr, edge_sh, W1, b1, W2, b2):
    edge_src = edge_index[0]
    edge_dst = edge_index[1]
    x = _sc_gather(node_attr, edge_dst)
    tp = _tc_edge(edge_attr, x, edge_sh, W1, b1, W2, b2)
    psum, pcnt = _sc_scatter(tp, edge_src)
    return _tc_combine(psum, pcnt, node_attr)


# speedup vs baseline: 4.4650x; 1.1196x over previous
import jax
import jax.numpy as jnp
import numpy as np
from jax import lax
from jax.experimental import pallas as pl
from jax.experimental.pallas import tpu as pltpu
from jax.experimental.pallas import tpu_sc as plsc

N_NODES = 10000
N_EDGES = 320000
F_IN = 16
F_HID = 32
F_W = 256
ALPHA = 1.0 / np.sqrt(16.0)

NC = 2
NS = 16
NW = NC * NS
EPW = N_EDGES // NW
CH = 2000
NCHUNK = EPW // CH
NPAD = 10240
STRIPE = NPAD // NS

def _mesh():
    return plsc.VectorSubcoreMesh(core_axis_name="c", subcore_axis_name="s",
                                  num_cores=NC, num_subcores=NS)


def _gather_body(nodes_hbm, dst_hbm, x_hbm,
                 idx0, idx1, rows0, rows1,
                 isem0, isem1, gsem, wsem0, wsem1):
    c = lax.axis_index("c")
    s = lax.axis_index("s")
    wid = s * NC + c
    base = wid * EPW
    idx = (idx0, idx1)
    rows = (rows0, rows1)
    isem = (isem0, isem1)
    wsem = (wsem0, wsem1)

    ih = [None] * NCHUNK
    wh = [None] * NCHUNK
    ih[0] = pltpu.async_copy(dst_hbm.at[pl.ds(base, CH)], idx[0], isem[0])
    for j in range(NCHUNK):
        b = j % 2
        if j + 1 < NCHUNK:
            off_n = base + (j + 1) * CH
            ih[j + 1] = pltpu.async_copy(dst_hbm.at[pl.ds(off_n, CH)],
                                         idx[1 - b], isem[1 - b])
        ih[j].wait()
        if j >= 2:
            wh[j - 2].wait()
        pltpu.async_copy(nodes_hbm.at[idx[b]], rows[b], gsem).wait()
        wh[j] = pltpu.async_copy(rows[b], x_hbm.at[pl.ds(base + j * CH, CH)],
                                 wsem[b])
    wh[NCHUNK - 2].wait()
    wh[NCHUNK - 1].wait()


@jax.jit
def _sc_gather(node_attr, edge_dst):
    return pl.kernel(
        _gather_body,
        out_type=jax.ShapeDtypeStruct((N_EDGES, F_IN), jnp.float32),
        mesh=_mesh(),
        compiler_params=pltpu.CompilerParams(use_tc_tiling_on_sc=False),
        scratch_types=[
            pltpu.VMEM((CH,), jnp.int32),
            pltpu.VMEM((CH,), jnp.int32),
            pltpu.VMEM((CH, F_IN), jnp.float32),
            pltpu.VMEM((CH, F_IN), jnp.float32),
            pltpu.SemaphoreType.DMA,
            pltpu.SemaphoreType.DMA,
            pltpu.SemaphoreType.DMA,
            pltpu.SemaphoreType.DMA,
            pltpu.SemaphoreType.DMA,
        ],
    )(node_attr, edge_dst)


def _scatter_body(tp_hbm, src_hbm, psum_hbm, pcnt_hbm,
                  idx0, idx1, rows0, rows1, ones_v, zrow_v, zcnt_v,
                  acc_sh, cnt_sh, isem0, isem1, rsem0, rsem1):
    c = lax.axis_index("c")
    s = lax.axis_index("s")
    wid = s * NC + c
    base = wid * EPW
    idx = (idx0, idx1)
    rows = (rows0, rows1)
    isem = (isem0, isem1)
    rsem = (rsem0, rsem1)

    ih = [None] * NCHUNK
    rh = [None] * NCHUNK
    ih[0] = pltpu.async_copy(src_hbm.at[pl.ds(base, CH)], idx[0], isem[0])
    rh[0] = pltpu.async_copy(tp_hbm.at[pl.ds(base, CH)], rows[0], rsem[0])

    onesv = jnp.ones((16,), jnp.float32)
    zerov = jnp.zeros((16,), jnp.float32)

    def fones(i, carry):
        ones_v[pl.ds(i * 16, 16)] = onesv
        return carry
    lax.fori_loop(0, CH // 16, fones, 0)

    def zrow(i, carry):
        zrow_v[i, :] = zerov
        return carry
    lax.fori_loop(0, STRIPE, zrow, 0)

    def zcnt(i, carry):
        zcnt_v[pl.ds(i * 16, 16)] = zerov
        return carry
    lax.fori_loop(0, STRIPE // 16, zcnt, 0)

    pltpu.sync_copy(zrow_v, acc_sh.at[pl.ds(s * STRIPE, STRIPE)])
    pltpu.sync_copy(zcnt_v, cnt_sh.at[pl.ds(s * STRIPE, STRIPE)])
    plsc.subcore_barrier()

    for j in range(NCHUNK):
        b = j % 2
        if j + 1 < NCHUNK:
            off_n = base + (j + 1) * CH
            ih[j + 1] = pltpu.async_copy(src_hbm.at[pl.ds(off_n, CH)],
                                         idx[1 - b], isem[1 - b])
            rh[j + 1] = pltpu.async_copy(tp_hbm.at[pl.ds(off_n, CH)],
                                         rows[1 - b], rsem[1 - b])
        ih[j].wait()
        rh[j].wait()
        pltpu.sync_copy(rows[b], acc_sh.at[idx[b]], add=True)
        pltpu.sync_copy(ones_v, cnt_sh.at[idx[b]], add=True)

    plsc.subcore_barrier()
    pltpu.sync_copy(acc_sh.at[pl.ds(s * STRIPE, STRIPE)],
                    psum_hbm.at[c, pl.ds(s * STRIPE, STRIPE)])
    pltpu.sync_copy(cnt_sh.at[pl.ds(s * STRIPE, STRIPE)],
                    pcnt_hbm.at[c, pl.ds(s * STRIPE, STRIPE)])


@jax.jit
def _sc_scatter(tp, edge_src):
    return pl.kernel(
        _scatter_body,
        out_type=(
            jax.ShapeDtypeStruct((NC, NPAD, F_IN), jnp.float32),
            jax.ShapeDtypeStruct((NC, NPAD), jnp.float32),
        ),
        mesh=_mesh(),
        compiler_params=pltpu.CompilerParams(use_tc_tiling_on_sc=False),
        scratch_types=[
            pltpu.VMEM((CH,), jnp.int32),
            pltpu.VMEM((CH,), jnp.int32),
            pltpu.VMEM((CH, F_IN), jnp.float32),
            pltpu.VMEM((CH, F_IN), jnp.float32),
            pltpu.VMEM((CH,), jnp.float32),
            pltpu.VMEM((STRIPE, F_IN), jnp.float32),
            pltpu.VMEM((STRIPE,), jnp.float32),
            pltpu.VMEM_SHARED((NPAD, F_IN), jnp.float32),
            pltpu.VMEM_SHARED((NPAD,), jnp.float32),
            pltpu.SemaphoreType.DMA,
            pltpu.SemaphoreType.DMA,
            pltpu.SemaphoreType.DMA,
            pltpu.SemaphoreType.DMA,
        ],
    )(tp, edge_src)


EB = 8000


def _edge_body(ea_ref, x_ref, y_ref, W1_ref, b1_ref, W2_ref, b2_ref,
               R_ref, S_ref, tp_ref):
    z = jnp.dot(ea_ref[...], W1_ref[...], preferred_element_type=jnp.float32)
    z = z + b1_ref[...][None, :]
    h = jnp.maximum(z, 0.0) + jnp.log1p(jnp.exp(-jnp.abs(z)))
    w = jnp.dot(h, W2_ref[...], preferred_element_type=jnp.float32)
    w = w + b2_ref[...][None, :]
    xr = jnp.dot(x_ref[...], R_ref[...], preferred_element_type=jnp.float32)
    tp = jnp.dot(xr * w, S_ref[...], preferred_element_type=jnp.float32)
    tp_ref[...] = tp * (ALPHA * y_ref[...])


@jax.jit
def _tc_edge(edge_attr, x, edge_sh, W1, b1, W2, b2):
    R = jnp.asarray(np.kron(np.eye(F_IN, dtype=np.float32),
                            np.ones((1, F_IN), np.float32)))
    S = jnp.asarray(np.kron(np.ones((F_IN, 1), np.float32),
                            np.eye(F_IN, dtype=np.float32)))
    grid = N_EDGES // EB
    return pl.pallas_call(
        _edge_body,
        grid=(grid,),
        in_specs=[
            pl.BlockSpec((EB, F_HID), lambda i: (i, 0)),
            pl.BlockSpec((EB, F_IN), lambda i: (i, 0)),
            pl.BlockSpec((EB, 1), lambda i: (i, 0)),
            pl.BlockSpec((F_HID, F_HID), lambda i: (0, 0)),
            pl.BlockSpec((F_HID,), lambda i: (0,)),
            pl.BlockSpec((F_HID, F_W), lambda i: (0, 0)),
            pl.BlockSpec((F_W,), lambda i: (0,)),
            pl.BlockSpec((F_IN, F_W), lambda i: (0, 0)),
            pl.BlockSpec((F_W, F_IN), lambda i: (0, 0)),
        ],
        out_specs=pl.BlockSpec((EB, F_IN), lambda i: (i, 0)),
        out_shape=jax.ShapeDtypeStruct((N_EDGES, F_IN), jnp.float32),
    )(edge_attr, x, edge_sh, W1, b1, W2, b2, R, S)


def _combine_body(p0_ref, p1_ref, c0_ref, c1_ref, na_ref, out_ref):
    cnt = jnp.maximum(c0_ref[...] + c1_ref[...], 1.0)
    out_ref[...] = (p0_ref[...] + p1_ref[...]) / cnt + na_ref[...]


@jax.jit
def _tc_combine(psum, pcnt, node_attr):
    p0 = psum[0, :N_NODES]
    p1 = psum[1, :N_NODES]
    c0 = pcnt[0, :N_NODES].reshape(N_NODES, 1)
    c1 = pcnt[1, :N_NODES].reshape(N_NODES, 1)
    return pl.pallas_call(
        _combine_body,
        out_shape=jax.ShapeDtypeStruct((N_NODES, F_IN), jnp.float32),
    )(p0, p1, c0, c1, node_attr)


def kernel(node_attr, edge_index, edge_attr, edge_sh, W1, b1, W2, b2):
    edge_src = edge_index[0]
    edge_dst = edge_index[1]
    x = _sc_gather(node_attr, edge_dst)
    tp = _tc_edge(edge_attr, x, edge_sh, W1, b1, W2, b2)
    psum, pcnt = _sc_scatter(tp, edge_src)
    return _tc_combine(psum, pcnt, node_attr)


# speedup vs baseline: 4.4865x; 1.0048x over previous
import jax
import jax.numpy as jnp
import numpy as np
from jax import lax
from jax.experimental import pallas as pl
from jax.experimental.pallas import tpu as pltpu
from jax.experimental.pallas import tpu_sc as plsc

N_NODES = 10000
N_EDGES = 320000
F_IN = 16
F_HID = 32
F_W = 256
ALPHA = 1.0 / np.sqrt(16.0)

NC = 2
NS = 16
NW = NC * NS
EPW = N_EDGES // NW
CH = 2000
NCHUNK = EPW // CH
NPAD = 10240
STRIPE = NPAD // NS

def _mesh():
    return plsc.VectorSubcoreMesh(core_axis_name="c", subcore_axis_name="s",
                                  num_cores=NC, num_subcores=NS)


def _gather_body(nodes_hbm, dst_hbm, x_hbm,
                 idx0, idx1, rows0, rows1,
                 isem0, isem1, gsem, wsem0, wsem1):
    c = lax.axis_index("c")
    s = lax.axis_index("s")
    wid = s * NC + c
    base = wid * EPW
    idx = (idx0, idx1)
    rows = (rows0, rows1)
    isem = (isem0, isem1)
    wsem = (wsem0, wsem1)

    ih = [None] * NCHUNK
    wh = [None] * NCHUNK
    ih[0] = pltpu.async_copy(dst_hbm.at[pl.ds(base, CH)], idx[0], isem[0])
    for j in range(NCHUNK):
        b = j % 2
        if j + 1 < NCHUNK:
            off_n = base + (j + 1) * CH
            ih[j + 1] = pltpu.async_copy(dst_hbm.at[pl.ds(off_n, CH)],
                                         idx[1 - b], isem[1 - b])
        ih[j].wait()
        if j >= 2:
            wh[j - 2].wait()
        pltpu.async_copy(nodes_hbm.at[idx[b]], rows[b], gsem).wait()
        wh[j] = pltpu.async_copy(rows[b], x_hbm.at[pl.ds(base + j * CH, CH)],
                                 wsem[b])
    wh[NCHUNK - 2].wait()
    wh[NCHUNK - 1].wait()


@jax.jit
def _sc_gather(node_attr, edge_dst):
    return pl.kernel(
        _gather_body,
        out_type=jax.ShapeDtypeStruct((N_EDGES, F_IN), jnp.float32),
        mesh=_mesh(),
        compiler_params=pltpu.CompilerParams(use_tc_tiling_on_sc=False),
        scratch_types=[
            pltpu.VMEM((CH,), jnp.int32),
            pltpu.VMEM((CH,), jnp.int32),
            pltpu.VMEM((CH, F_IN), jnp.float32),
            pltpu.VMEM((CH, F_IN), jnp.float32),
            pltpu.SemaphoreType.DMA,
            pltpu.SemaphoreType.DMA,
            pltpu.SemaphoreType.DMA,
            pltpu.SemaphoreType.DMA,
            pltpu.SemaphoreType.DMA,
        ],
    )(node_attr, edge_dst)


def _scatter_body(tp_hbm, src_hbm, psum_hbm, pcnt_hbm,
                  idx0, idx1, rows0, rows1, ones_v, zrow_v, zcnt_v,
                  acc_sh, cnt_sh, isem0, isem1, rsem0, rsem1):
    c = lax.axis_index("c")
    s = lax.axis_index("s")
    wid = s * NC + c
    base = wid * EPW
    idx = (idx0, idx1)
    rows = (rows0, rows1)
    isem = (isem0, isem1)
    rsem = (rsem0, rsem1)

    ih = [None] * NCHUNK
    rh = [None] * NCHUNK
    ih[0] = pltpu.async_copy(src_hbm.at[pl.ds(base, CH)], idx[0], isem[0])
    rh[0] = pltpu.async_copy(tp_hbm.at[pl.ds(base, CH)], rows[0], rsem[0])

    onesv = jnp.ones((16,), jnp.float32)
    zerov = jnp.zeros((16,), jnp.float32)

    def fones(i, carry):
        ones_v[pl.ds(i * 16, 16)] = onesv
        return carry
    lax.fori_loop(0, CH // 16, fones, 0)

    def zrow(i, carry):
        zrow_v[i, :] = zerov
        return carry
    lax.fori_loop(0, STRIPE, zrow, 0)

    def zcnt(i, carry):
        zcnt_v[pl.ds(i * 16, 16)] = zerov
        return carry
    lax.fori_loop(0, STRIPE // 16, zcnt, 0)

    pltpu.sync_copy(zrow_v, acc_sh.at[pl.ds(s * STRIPE, STRIPE)])
    pltpu.sync_copy(zcnt_v, cnt_sh.at[pl.ds(s * STRIPE, STRIPE)])
    plsc.subcore_barrier()

    for j in range(NCHUNK):
        b = j % 2
        if j + 1 < NCHUNK:
            off_n = base + (j + 1) * CH
            ih[j + 1] = pltpu.async_copy(src_hbm.at[pl.ds(off_n, CH)],
                                         idx[1 - b], isem[1 - b])
            rh[j + 1] = pltpu.async_copy(tp_hbm.at[pl.ds(off_n, CH)],
                                         rows[1 - b], rsem[1 - b])
        ih[j].wait()
        rh[j].wait()
        pltpu.sync_copy(rows[b], acc_sh.at[idx[b]], add=True)
        pltpu.sync_copy(ones_v, cnt_sh.at[idx[b]], add=True)

    plsc.subcore_barrier()
    pltpu.sync_copy(acc_sh.at[pl.ds(s * STRIPE, STRIPE)],
                    psum_hbm.at[c, pl.ds(s * STRIPE, STRIPE)])
    pltpu.sync_copy(cnt_sh.at[pl.ds(s * STRIPE, STRIPE)],
                    pcnt_hbm.at[c, pl.ds(s * STRIPE, STRIPE)])


@jax.jit
def _sc_scatter(tp, edge_src):
    return pl.kernel(
        _scatter_body,
        out_type=(
            jax.ShapeDtypeStruct((NC, NPAD, F_IN), jnp.float32),
            jax.ShapeDtypeStruct((NC, NPAD), jnp.float32),
        ),
        mesh=_mesh(),
        compiler_params=pltpu.CompilerParams(use_tc_tiling_on_sc=False),
        scratch_types=[
            pltpu.VMEM((CH,), jnp.int32),
            pltpu.VMEM((CH,), jnp.int32),
            pltpu.VMEM((CH, F_IN), jnp.float32),
            pltpu.VMEM((CH, F_IN), jnp.float32),
            pltpu.VMEM((CH,), jnp.float32),
            pltpu.VMEM((STRIPE, F_IN), jnp.float32),
            pltpu.VMEM((STRIPE,), jnp.float32),
            pltpu.VMEM_SHARED((NPAD, F_IN), jnp.float32),
            pltpu.VMEM_SHARED((NPAD,), jnp.float32),
            pltpu.SemaphoreType.DMA,
            pltpu.SemaphoreType.DMA,
            pltpu.SemaphoreType.DMA,
            pltpu.SemaphoreType.DMA,
        ],
    )(tp, edge_src)


EB = 10000


def _edge_body(ea_ref, x_ref, y_ref, W1_ref, b1_ref, W2_ref, b2_ref,
               R_ref, S_ref, tp_ref):
    z = jnp.dot(ea_ref[...], W1_ref[...], preferred_element_type=jnp.float32)
    z = z + b1_ref[...][None, :]
    h = jnp.maximum(z, 0.0) + jnp.log1p(jnp.exp(-jnp.abs(z)))
    w = jnp.dot(h, W2_ref[...], preferred_element_type=jnp.float32)
    w = w + b2_ref[...][None, :]
    xr = jnp.dot(x_ref[...], R_ref[...], preferred_element_type=jnp.float32)
    tp = jnp.dot(xr * w, S_ref[...], preferred_element_type=jnp.float32)
    tp_ref[...] = tp * (ALPHA * y_ref[...])


@jax.jit
def _tc_edge(edge_attr, x, edge_sh, W1, b1, W2, b2):
    R = jnp.asarray(np.kron(np.eye(F_IN, dtype=np.float32),
                            np.ones((1, F_IN), np.float32)))
    S = jnp.asarray(np.kron(np.ones((F_IN, 1), np.float32),
                            np.eye(F_IN, dtype=np.float32)))
    grid = N_EDGES // EB
    return pl.pallas_call(
        _edge_body,
        grid=(grid,),
        in_specs=[
            pl.BlockSpec((EB, F_HID), lambda i: (i, 0)),
            pl.BlockSpec((EB, F_IN), lambda i: (i, 0)),
            pl.BlockSpec((EB, 1), lambda i: (i, 0)),
            pl.BlockSpec((F_HID, F_HID), lambda i: (0, 0)),
            pl.BlockSpec((F_HID,), lambda i: (0,)),
            pl.BlockSpec((F_HID, F_W), lambda i: (0, 0)),
            pl.BlockSpec((F_W,), lambda i: (0,)),
            pl.BlockSpec((F_IN, F_W), lambda i: (0, 0)),
            pl.BlockSpec((F_W, F_IN), lambda i: (0, 0)),
        ],
        out_specs=pl.BlockSpec((EB, F_IN), lambda i: (i, 0)),
        out_shape=jax.ShapeDtypeStruct((N_EDGES, F_IN), jnp.float32),
    )(edge_attr, x, edge_sh, W1, b1, W2, b2, R, S)


def _combine_body(p0_ref, p1_ref, c0_ref, c1_ref, na_ref, out_ref):
    cnt = jnp.maximum(c0_ref[...] + c1_ref[...], 1.0)
    out_ref[...] = (p0_ref[...] + p1_ref[...]) / cnt + na_ref[...]


@jax.jit
def _tc_combine(psum, pcnt, node_attr):
    p0 = psum[0, :N_NODES]
    p1 = psum[1, :N_NODES]
    c0 = pcnt[0, :N_NODES].reshape(N_NODES, 1)
    c1 = pcnt[1, :N_NODES].reshape(N_NODES, 1)
    return pl.pallas_call(
        _combine_body,
        out_shape=jax.ShapeDtypeStruct((N_NODES, F_IN), jnp.float32),
    )(p0, p1, c0, c1, node_attr)


def kernel(node_attr, edge_index, edge_attr, edge_sh, W1, b1, W2, b2):
    edge_src = edge_index[0]
    edge_dst = edge_index[1]
    x = _sc_gather(node_attr, edge_dst)
    tp = _tc_edge(edge_attr, x, edge_sh, W1, b1, W2, b2)
    psum, pcnt = _sc_scatter(tp, edge_src)
    return _tc_combine(psum, pcnt, node_attr)


# speedup vs baseline: 7.0376x; 1.5686x over previous
import jax
import jax.numpy as jnp
import numpy as np
from jax import lax
from jax.experimental import pallas as pl
from jax.experimental.pallas import tpu as pltpu
from jax.experimental.pallas import tpu_sc as plsc

N_NODES = 10000
N_EDGES = 320000
F_IN = 16
F_HID = 32
F_W = 256
ALPHA = 1.0 / np.sqrt(16.0)

NC = 2
NS = 16
NW = NC * NS
EPW = N_EDGES // NW
CH = 2000
NCHUNK = EPW // CH
NPAD = 10240
STRIPE = NPAD // NS

def _mesh():
    return plsc.VectorSubcoreMesh(core_axis_name="c", subcore_axis_name="s",
                                  num_cores=NC, num_subcores=NS)


def _gather_body(nodes_hbm, dst_hbm, x_hbm,
                 idx0, idx1, rows0, rows1,
                 isem0, isem1, gsem, wsem0, wsem1):
    c = lax.axis_index("c")
    s = lax.axis_index("s")
    wid = s * NC + c
    base = wid * EPW
    idx = (idx0, idx1)
    rows = (rows0, rows1)
    isem = (isem0, isem1)
    wsem = (wsem0, wsem1)

    ih = [None] * NCHUNK
    wh = [None] * NCHUNK
    ih[0] = pltpu.async_copy(dst_hbm.at[pl.ds(base, CH)], idx[0], isem[0])
    for j in range(NCHUNK):
        b = j % 2
        if j + 1 < NCHUNK:
            off_n = base + (j + 1) * CH
            ih[j + 1] = pltpu.async_copy(dst_hbm.at[pl.ds(off_n, CH)],
                                         idx[1 - b], isem[1 - b])
        ih[j].wait()
        if j >= 2:
            wh[j - 2].wait()
        pltpu.async_copy(nodes_hbm.at[idx[b]], rows[b], gsem).wait()
        wh[j] = pltpu.async_copy(rows[b], x_hbm.at[pl.ds(base + j * CH, CH)],
                                 wsem[b])
    wh[NCHUNK - 2].wait()
    wh[NCHUNK - 1].wait()


@jax.jit
def _sc_gather(node_attr, edge_dst):
    return pl.kernel(
        _gather_body,
        out_type=jax.ShapeDtypeStruct((N_EDGES, F_IN), jnp.float32),
        mesh=_mesh(),
        compiler_params=pltpu.CompilerParams(use_tc_tiling_on_sc=False),
        scratch_types=[
            pltpu.VMEM((CH,), jnp.int32),
            pltpu.VMEM((CH,), jnp.int32),
            pltpu.VMEM((CH, F_IN), jnp.float32),
            pltpu.VMEM((CH, F_IN), jnp.float32),
            pltpu.SemaphoreType.DMA,
            pltpu.SemaphoreType.DMA,
            pltpu.SemaphoreType.DMA,
            pltpu.SemaphoreType.DMA,
            pltpu.SemaphoreType.DMA,
        ],
    )(node_attr, edge_dst)


def _scatter_body(tp_hbm, src_hbm, psum_hbm, pcnt_hbm,
                  idx0, idx1, rows0, rows1, ones_v, zrow_v, zcnt_v,
                  acc_sh, cnt_sh, isem0, isem1, rsem0, rsem1):
    c = lax.axis_index("c")
    s = lax.axis_index("s")
    wid = s * NC + c
    base = wid * EPW
    idx = (idx0, idx1)
    rows = (rows0, rows1)
    isem = (isem0, isem1)
    rsem = (rsem0, rsem1)

    ih = [None] * NCHUNK
    rh = [None] * NCHUNK
    ih[0] = pltpu.async_copy(src_hbm.at[pl.ds(base, CH)], idx[0], isem[0])
    rh[0] = pltpu.async_copy(tp_hbm.at[pl.ds(base, CH)], rows[0], rsem[0])

    onesv = jnp.ones((16,), jnp.float32)
    zerov = jnp.zeros((16,), jnp.float32)

    def fones(i, carry):
        ones_v[pl.ds(i * 16, 16)] = onesv
        return carry
    lax.fori_loop(0, CH // 16, fones, 0)

    def zrow(i, carry):
        zrow_v[i, :] = zerov
        return carry
    lax.fori_loop(0, STRIPE, zrow, 0)

    def zcnt(i, carry):
        zcnt_v[pl.ds(i * 16, 16)] = zerov
        return carry
    lax.fori_loop(0, STRIPE // 16, zcnt, 0)

    pltpu.sync_copy(zrow_v, acc_sh.at[pl.ds(s * STRIPE, STRIPE)])
    pltpu.sync_copy(zcnt_v, cnt_sh.at[pl.ds(s * STRIPE, STRIPE)])
    plsc.subcore_barrier()

    for j in range(NCHUNK):
        b = j % 2
        if j + 1 < NCHUNK:
            off_n = base + (j + 1) * CH
            ih[j + 1] = pltpu.async_copy(src_hbm.at[pl.ds(off_n, CH)],
                                         idx[1 - b], isem[1 - b])
            rh[j + 1] = pltpu.async_copy(tp_hbm.at[pl.ds(off_n, CH)],
                                         rows[1 - b], rsem[1 - b])
        ih[j].wait()
        rh[j].wait()
        pltpu.sync_copy(rows[b], acc_sh.at[idx[b]], add=True)
        pltpu.sync_copy(ones_v, cnt_sh.at[idx[b]], add=True)

    plsc.subcore_barrier()
    pltpu.sync_copy(acc_sh.at[pl.ds(s * STRIPE, STRIPE)],
                    psum_hbm.at[c, pl.ds(s * STRIPE, STRIPE)])
    pltpu.sync_copy(cnt_sh.at[pl.ds(s * STRIPE, STRIPE)],
                    pcnt_hbm.at[c, pl.ds(s * STRIPE, STRIPE)])


@jax.jit
def _sc_scatter(tp, edge_src):
    return pl.kernel(
        _scatter_body,
        out_type=(
            jax.ShapeDtypeStruct((NC, NPAD, F_IN), jnp.float32),
            jax.ShapeDtypeStruct((NC, NPAD), jnp.float32),
        ),
        mesh=_mesh(),
        compiler_params=pltpu.CompilerParams(use_tc_tiling_on_sc=False),
        scratch_types=[
            pltpu.VMEM((CH,), jnp.int32),
            pltpu.VMEM((CH,), jnp.int32),
            pltpu.VMEM((CH, F_IN), jnp.float32),
            pltpu.VMEM((CH, F_IN), jnp.float32),
            pltpu.VMEM((CH,), jnp.float32),
            pltpu.VMEM((STRIPE, F_IN), jnp.float32),
            pltpu.VMEM((STRIPE,), jnp.float32),
            pltpu.VMEM_SHARED((NPAD, F_IN), jnp.float32),
            pltpu.VMEM_SHARED((NPAD,), jnp.float32),
            pltpu.SemaphoreType.DMA,
            pltpu.SemaphoreType.DMA,
            pltpu.SemaphoreType.DMA,
            pltpu.SemaphoreType.DMA,
        ],
    )(tp, edge_src)


PK = 8
BR = 1000
NROW = N_EDGES // PK
EA_W = PK * F_HID
X_W = PK * F_IN
W_W = PK * F_W


def _edge_body(ea_ref, x_ref, y_ref, W1_ref, b1_ref, W2_ref, b2_ref,
               R_ref, S_ref, K_ref, tp_ref):
    z = jnp.dot(ea_ref[...], W1_ref[...], preferred_element_type=jnp.float32)
    z = z + b1_ref[...][None, :]
    h = jnp.maximum(z, 0.0) + jnp.log1p(jnp.exp(-jnp.abs(z)))
    w = jnp.dot(h, W2_ref[...], preferred_element_type=jnp.float32)
    w = w + b2_ref[...][None, :]
    xr = jnp.dot(x_ref[...], R_ref[...], preferred_element_type=jnp.float32)
    tp = jnp.dot(xr * w, S_ref[...], preferred_element_type=jnp.float32)
    yexp = jnp.dot(y_ref[...], K_ref[...], preferred_element_type=jnp.float32)
    tp_ref[...] = tp * (ALPHA * yexp)


@jax.jit
def _tc_edge(edge_attr, x, edge_sh, W1, b1, W2, b2):
    eye8 = np.eye(PK, dtype=np.float32)
    R = np.kron(np.eye(F_IN, dtype=np.float32), np.ones((1, F_IN), np.float32))
    S = np.kron(np.ones((F_IN, 1), np.float32), np.eye(F_IN, dtype=np.float32))
    R8 = jnp.asarray(np.kron(eye8, R))
    S8 = jnp.asarray(np.kron(eye8, S))
    K8 = jnp.asarray(np.kron(eye8, np.ones((1, F_IN), np.float32)))
    W1bd = jnp.kron(jnp.asarray(eye8), W1)
    W2bd = jnp.kron(jnp.asarray(eye8), W2)
    b1t = jnp.tile(b1, PK)
    b2t = jnp.tile(b2, PK)
    ea8 = edge_attr.reshape(NROW, EA_W)
    x8 = x.reshape(NROW, X_W)
    y8 = edge_sh.reshape(NROW, PK)
    grid = NROW // BR
    tp8 = pl.pallas_call(
        _edge_body,
        grid=(grid,),
        in_specs=[
            pl.BlockSpec((BR, EA_W), lambda i: (i, 0)),
            pl.BlockSpec((BR, X_W), lambda i: (i, 0)),
            pl.BlockSpec((BR, PK), lambda i: (i, 0)),
            pl.BlockSpec((EA_W, EA_W), lambda i: (0, 0)),
            pl.BlockSpec((EA_W,), lambda i: (0,)),
            pl.BlockSpec((EA_W, W_W), lambda i: (0, 0)),
            pl.BlockSpec((W_W,), lambda i: (0,)),
            pl.BlockSpec((X_W, W_W), lambda i: (0, 0)),
            pl.BlockSpec((W_W, X_W), lambda i: (0, 0)),
            pl.BlockSpec((PK, X_W), lambda i: (0, 0)),
        ],
        out_specs=pl.BlockSpec((BR, X_W), lambda i: (i, 0)),
        out_shape=jax.ShapeDtypeStruct((NROW, X_W), jnp.float32),
    )(ea8, x8, y8, W1bd, b1t, W2bd, b2t, R8, S8, K8)
    return tp8.reshape(N_EDGES, F_IN)


def _combine_body(p0_ref, p1_ref, c0_ref, c1_ref, na_ref, out_ref):
    cnt = jnp.maximum(c0_ref[...] + c1_ref[...], 1.0)
    out_ref[...] = (p0_ref[...] + p1_ref[...]) / cnt + na_ref[...]


@jax.jit
def _tc_combine(psum, pcnt, node_attr):
    p0 = psum[0, :N_NODES]
    p1 = psum[1, :N_NODES]
    c0 = pcnt[0, :N_NODES].reshape(N_NODES, 1)
    c1 = pcnt[1, :N_NODES].reshape(N_NODES, 1)
    return pl.pallas_call(
        _combine_body,
        out_shape=jax.ShapeDtypeStruct((N_NODES, F_IN), jnp.float32),
    )(p0, p1, c0, c1, node_attr)


def kernel(node_attr, edge_index, edge_attr, edge_sh, W1, b1, W2, b2):
    edge_src = edge_index[0]
    edge_dst = edge_index[1]
    x = _sc_gather(node_attr, edge_dst)
    tp = _tc_edge(edge_attr, x, edge_sh, W1, b1, W2, b2)
    psum, pcnt = _sc_scatter(tp, edge_src)
    return _tc_combine(psum, pcnt, node_attr)


# speedup vs baseline: 7.0926x; 1.0078x over previous
import jax
import jax.numpy as jnp
import numpy as np
from jax import lax
from jax.experimental import pallas as pl
from jax.experimental.pallas import tpu as pltpu
from jax.experimental.pallas import tpu_sc as plsc

N_NODES = 10000
N_EDGES = 320000
F_IN = 16
F_HID = 32
F_W = 256
ALPHA = 1.0 / np.sqrt(16.0)

NC = 2
NS = 16
NW = NC * NS
EPW = N_EDGES // NW
CH = 2000
NCHUNK = EPW // CH
NPAD = 10240
STRIPE = NPAD // NS

def _mesh():
    return plsc.VectorSubcoreMesh(core_axis_name="c", subcore_axis_name="s",
                                  num_cores=NC, num_subcores=NS)


def _gather_body(nodes_hbm, dst_hbm, x_hbm,
                 idx0, idx1, rows0, rows1,
                 isem0, isem1, gsem, wsem0, wsem1):
    c = lax.axis_index("c")
    s = lax.axis_index("s")
    wid = s * NC + c
    base = wid * EPW
    idx = (idx0, idx1)
    rows = (rows0, rows1)
    isem = (isem0, isem1)
    wsem = (wsem0, wsem1)

    ih = [None] * NCHUNK
    wh = [None] * NCHUNK
    ih[0] = pltpu.async_copy(dst_hbm.at[pl.ds(base, CH)], idx[0], isem[0])
    for j in range(NCHUNK):
        b = j % 2
        if j + 1 < NCHUNK:
            off_n = base + (j + 1) * CH
            ih[j + 1] = pltpu.async_copy(dst_hbm.at[pl.ds(off_n, CH)],
                                         idx[1 - b], isem[1 - b])
        ih[j].wait()
        if j >= 2:
            wh[j - 2].wait()
        pltpu.async_copy(nodes_hbm.at[idx[b]], rows[b], gsem).wait()
        wh[j] = pltpu.async_copy(rows[b], x_hbm.at[pl.ds(base + j * CH, CH)],
                                 wsem[b])
    wh[NCHUNK - 2].wait()
    wh[NCHUNK - 1].wait()


@jax.jit
def _sc_gather(node_attr, edge_dst):
    return pl.kernel(
        _gather_body,
        out_type=jax.ShapeDtypeStruct((N_EDGES, F_IN), jnp.float32),
        mesh=_mesh(),
        compiler_params=pltpu.CompilerParams(use_tc_tiling_on_sc=False),
        scratch_types=[
            pltpu.VMEM((CH,), jnp.int32),
            pltpu.VMEM((CH,), jnp.int32),
            pltpu.VMEM((CH, F_IN), jnp.float32),
            pltpu.VMEM((CH, F_IN), jnp.float32),
            pltpu.SemaphoreType.DMA,
            pltpu.SemaphoreType.DMA,
            pltpu.SemaphoreType.DMA,
            pltpu.SemaphoreType.DMA,
            pltpu.SemaphoreType.DMA,
        ],
    )(node_attr, edge_dst)


def _scatter_body(tp_hbm, src_hbm, psum_hbm, pcnt_hbm,
                  idx0, idx1, rows0, rows1, ones_v, zrow_v, zcnt_v,
                  acc_sh, cnt_sh, isem0, isem1, rsem0, rsem1):
    c = lax.axis_index("c")
    s = lax.axis_index("s")
    wid = s * NC + c
    base = wid * EPW
    idx = (idx0, idx1)
    rows = (rows0, rows1)
    isem = (isem0, isem1)
    rsem = (rsem0, rsem1)

    ih = [None] * NCHUNK
    rh = [None] * NCHUNK
    ih[0] = pltpu.async_copy(src_hbm.at[pl.ds(base, CH)], idx[0], isem[0])
    rh[0] = pltpu.async_copy(tp_hbm.at[pl.ds(base, CH)], rows[0], rsem[0])

    onesv = jnp.ones((16,), jnp.float32)
    zerov = jnp.zeros((16,), jnp.float32)

    def fones(i, carry):
        ones_v[pl.ds(i * 16, 16)] = onesv
        return carry
    lax.fori_loop(0, CH // 16, fones, 0)

    def zrow(i, carry):
        zrow_v[i, :] = zerov
        return carry
    lax.fori_loop(0, STRIPE, zrow, 0)

    def zcnt(i, carry):
        zcnt_v[pl.ds(i * 16, 16)] = zerov
        return carry
    lax.fori_loop(0, STRIPE // 16, zcnt, 0)

    pltpu.sync_copy(zrow_v, acc_sh.at[pl.ds(s * STRIPE, STRIPE)])
    pltpu.sync_copy(zcnt_v, cnt_sh.at[pl.ds(s * STRIPE, STRIPE)])
    plsc.subcore_barrier()

    for j in range(NCHUNK):
        b = j % 2
        if j + 1 < NCHUNK:
            off_n = base + (j + 1) * CH
            ih[j + 1] = pltpu.async_copy(src_hbm.at[pl.ds(off_n, CH)],
                                         idx[1 - b], isem[1 - b])
            rh[j + 1] = pltpu.async_copy(tp_hbm.at[pl.ds(off_n, CH)],
                                         rows[1 - b], rsem[1 - b])
        ih[j].wait()
        rh[j].wait()
        pltpu.sync_copy(rows[b], acc_sh.at[idx[b]], add=True)
        pltpu.sync_copy(ones_v, cnt_sh.at[idx[b]], add=True)

    plsc.subcore_barrier()
    pltpu.sync_copy(acc_sh.at[pl.ds(s * STRIPE, STRIPE)],
                    psum_hbm.at[c, pl.ds(s * STRIPE, STRIPE)])
    pltpu.sync_copy(cnt_sh.at[pl.ds(s * STRIPE, STRIPE)],
                    pcnt_hbm.at[c, pl.ds(s * STRIPE, STRIPE)])


@jax.jit
def _sc_scatter(tp, edge_src):
    return pl.kernel(
        _scatter_body,
        out_type=(
            jax.ShapeDtypeStruct((NC, NPAD, F_IN), jnp.float32),
            jax.ShapeDtypeStruct((NC, NPAD), jnp.float32),
        ),
        mesh=_mesh(),
        compiler_params=pltpu.CompilerParams(use_tc_tiling_on_sc=False),
        scratch_types=[
            pltpu.VMEM((CH,), jnp.int32),
            pltpu.VMEM((CH,), jnp.int32),
            pltpu.VMEM((CH, F_IN), jnp.float32),
            pltpu.VMEM((CH, F_IN), jnp.float32),
            pltpu.VMEM((CH,), jnp.float32),
            pltpu.VMEM((STRIPE, F_IN), jnp.float32),
            pltpu.VMEM((STRIPE,), jnp.float32),
            pltpu.VMEM_SHARED((NPAD, F_IN), jnp.float32),
            pltpu.VMEM_SHARED((NPAD,), jnp.float32),
            pltpu.SemaphoreType.DMA,
            pltpu.SemaphoreType.DMA,
            pltpu.SemaphoreType.DMA,
            pltpu.SemaphoreType.DMA,
        ],
    )(tp, edge_src)


PK = 8
BR = 2000
NROW = N_EDGES // PK
EA_W = PK * F_HID
X_W = PK * F_IN
W_W = PK * F_W


def _edge_body(ea_ref, x_ref, y_ref, W1_ref, b1_ref, W2_ref, b2_ref,
               R_ref, S_ref, K_ref, tp_ref):
    z = jnp.dot(ea_ref[...], W1_ref[...], preferred_element_type=jnp.float32)
    z = z + b1_ref[...][None, :]
    h = jnp.maximum(z, 0.0) + jnp.log1p(jnp.exp(-jnp.abs(z)))
    w = jnp.dot(h, W2_ref[...], preferred_element_type=jnp.float32)
    w = w + b2_ref[...][None, :]
    xr = jnp.dot(x_ref[...], R_ref[...], preferred_element_type=jnp.float32)
    tp = jnp.dot(xr * w, S_ref[...], preferred_element_type=jnp.float32)
    yexp = jnp.dot(y_ref[...], K_ref[...], preferred_element_type=jnp.float32)
    tp_ref[...] = tp * (ALPHA * yexp)


@jax.jit
def _tc_edge(edge_attr, x, edge_sh, W1, b1, W2, b2):
    eye8 = np.eye(PK, dtype=np.float32)
    R = np.kron(np.eye(F_IN, dtype=np.float32), np.ones((1, F_IN), np.float32))
    S = np.kron(np.ones((F_IN, 1), np.float32), np.eye(F_IN, dtype=np.float32))
    R8 = jnp.asarray(np.kron(eye8, R))
    S8 = jnp.asarray(np.kron(eye8, S))
    K8 = jnp.asarray(np.kron(eye8, np.ones((1, F_IN), np.float32)))
    W1bd = jnp.kron(jnp.asarray(eye8), W1)
    W2bd = jnp.kron(jnp.asarray(eye8), W2)
    b1t = jnp.tile(b1, PK)
    b2t = jnp.tile(b2, PK)
    ea8 = edge_attr.reshape(NROW, EA_W)
    x8 = x.reshape(NROW, X_W)
    y8 = edge_sh.reshape(NROW, PK)
    grid = NROW // BR
    tp8 = pl.pallas_call(
        _edge_body,
        grid=(grid,),
        in_specs=[
            pl.BlockSpec((BR, EA_W), lambda i: (i, 0)),
            pl.BlockSpec((BR, X_W), lambda i: (i, 0)),
            pl.BlockSpec((BR, PK), lambda i: (i, 0)),
            pl.BlockSpec((EA_W, EA_W), lambda i: (0, 0)),
            pl.BlockSpec((EA_W,), lambda i: (0,)),
            pl.BlockSpec((EA_W, W_W), lambda i: (0, 0)),
            pl.BlockSpec((W_W,), lambda i: (0,)),
            pl.BlockSpec((X_W, W_W), lambda i: (0, 0)),
            pl.BlockSpec((W_W, X_W), lambda i: (0, 0)),
            pl.BlockSpec((PK, X_W), lambda i: (0, 0)),
        ],
        out_specs=pl.BlockSpec((BR, X_W), lambda i: (i, 0)),
        out_shape=jax.ShapeDtypeStruct((NROW, X_W), jnp.float32),
    )(ea8, x8, y8, W1bd, b1t, W2bd, b2t, R8, S8, K8)
    return tp8.reshape(N_EDGES, F_IN)


def _combine_body(p0_ref, p1_ref, c0_ref, c1_ref, na_ref, out_ref):
    cnt = jnp.maximum(c0_ref[...] + c1_ref[...], 1.0)
    out_ref[...] = (p0_ref[...] + p1_ref[...]) / cnt + na_ref[...]


@jax.jit
def _tc_combine(psum, pcnt, node_attr):
    p0 = psum[0, :N_NODES]
    p1 = psum[1, :N_NODES]
    c0 = pcnt[0, :N_NODES].reshape(N_NODES, 1)
    c1 = pcnt[1, :N_NODES].reshape(N_NODES, 1)
    return pl.pallas_call(
        _combine_body,
        out_shape=jax.ShapeDtypeStruct((N_NODES, F_IN), jnp.float32),
    )(p0, p1, c0, c1, node_attr)


def kernel(node_attr, edge_index, edge_attr, edge_sh, W1, b1, W2, b2):
    edge_src = edge_index[0]
    edge_dst = edge_index[1]
    x = _sc_gather(node_attr, edge_dst)
    tp = _tc_edge(edge_attr, x, edge_sh, W1, b1, W2, b2)
    psum, pcnt = _sc_scatter(tp, edge_src)
    return _tc_combine(psum, pcnt, node_attr)
